# baseline (speedup 1.0000x reference)
import jax
import jax.numpy as jnp
from jax import lax
from jax.experimental import pallas as pl
from jax.experimental.pallas import tpu as pltpu
from jax.experimental.pallas import tpu_sc as plsc

_HIDDEN = 128
_NG = 16
_LV = 102
_DV = 11
_NN = 32768
_NREL = 18

_NTILES = 32
_NPT = _NN // _NTILES
_NBINS = 4096


def _sc_hist_body(x_hbm, bat_hbm, cnt_hbm, x_v, bat_v, tab_v):
    cid = lax.axis_index("c")
    sid = lax.axis_index("s")
    wid = cid * 16 + sid
    base = wid * _NPT
    for k in range(_NBINS // 16):
        tab_v[pl.ds(k * 16, 16)] = jnp.zeros((16,), jnp.float32)
    pltpu.sync_copy(x_hbm.at[pl.ds(base * 2, _NPT * 2)], x_v)
    pltpu.sync_copy(bat_hbm.at[pl.ds(base, _NPT)], bat_v)
    ones = jnp.ones((16,), jnp.float32)
    iota2 = lax.iota(jnp.int32, 16) * 2
    for i in range(_NPT // 16):
        b16 = bat_v[pl.ds(i * 16, 16)] * 128
        l16 = jnp.clip(plsc.load_gather(x_v, [iota2 + (i * 32)]), 0, _LV - 1)
        d16 = jnp.clip(plsc.load_gather(x_v, [iota2 + (i * 32 + 1)]), 0, _DV - 1)
        plsc.addupdate_scatter(tab_v, [b16 + l16], ones)
        plsc.addupdate_scatter(tab_v, [b16 + d16 + 2048], ones)
    pltpu.sync_copy(tab_v, cnt_hbm.at[wid])


def _sc_hist(x_flat, bat):
    mesh = plsc.VectorSubcoreMesh(core_axis_name="c", subcore_axis_name="s",
                                  num_cores=2, num_subcores=16)
    return pl.kernel(
        _sc_hist_body,
        out_type=jax.ShapeDtypeStruct((_NTILES, _NBINS), jnp.float32),
        mesh=mesh,
        compiler_params=pltpu.CompilerParams(needs_layout_passes=False),
        scratch_types=[
            pltpu.VMEM((_NPT * 2,), jnp.int32),
            pltpu.VMEM((_NPT,), jnp.int32),
            pltpu.VMEM((_NBINS,), jnp.float32),
        ],
    )(x_flat, bat)


def _tc_head_body(cnt_ref, gf_ref, le_ref, de_ref, gw_ref, gb_ref,
                  w1_ref, b1_ref, w2_ref, b2_ref, out_ref):
    c = jnp.sum(cnt_ref[...], axis=0)
    counts_lab = c[:_NG, :_LV]
    counts_deg = c[_NG:, :_DV]
    hg = (jnp.dot(counts_lab, le_ref[...], preferred_element_type=jnp.float32,
                  precision=lax.Precision.HIGHEST)
          + jnp.dot(counts_deg, de_ref[...], preferred_element_type=jnp.float32,
                    precision=lax.Precision.HIGHEST))
    gp = jnp.dot(gf_ref[...], gw_ref[...], preferred_element_type=jnp.float32,
                 precision=lax.Precision.HIGHEST) + gb_ref[...]
    cat = jnp.concatenate([hg, gp], axis=1)
    hidden = jnp.maximum(
        jnp.dot(cat, w1_ref[...], preferred_element_type=jnp.float32,
                precision=lax.Precision.HIGHEST) + b1_ref[...], 0.0)
    out_ref[...] = jnp.dot(hidden, w2_ref[...], preferred_element_type=jnp.float32,
                           precision=lax.Precision.HIGHEST) + b2_ref[...]


def kernel(x, batch, g_feat, label_emb, deg_emb, gproj_w, gproj_b,
           w1, b1, w2, b2):
    x_flat = x.astype(jnp.int32).reshape(_NN * 2)
    bat = batch.astype(jnp.int32)
    counts = _sc_hist(x_flat, bat).reshape(_NTILES, 32, 128)
    out = pl.pallas_call(
        _tc_head_body,
        out_shape=jax.ShapeDtypeStruct((_NG, _NREL), jnp.float32),
    )(counts, g_feat, label_emb, deg_emb, gproj_w, gproj_b.reshape(1, _HIDDEN),
      w1, b1.reshape(1, _HIDDEN), w2, b2.reshape(1, _NREL))
    return out


# speedup vs baseline: 1.6330x; 1.6330x over previous
import jax
import jax.numpy as jnp
from jax import lax
from jax.experimental import pallas as pl
from jax.experimental.pallas import tpu as pltpu
from jax.experimental.pallas import tpu_sc as plsc

_HIDDEN = 128
_NG = 16
_LV = 102
_DV = 11
_NN = 32768
_NREL = 18

_NTILES = 32
_NPT = _NN // _NTILES
_NBINS = 4096


def _sc_hist_body(lab_hbm, deg_hbm, bat_hbm, cnt_hbm, lab_v, deg_v, bat_v, tab_v):
    cid = lax.axis_index("c")
    sid = lax.axis_index("s")
    wid = cid * 16 + sid
    base = wid * _NPT
    for k in range(_NBINS // 16):
        tab_v[pl.ds(k * 16, 16)] = jnp.zeros((16,), jnp.float32)
    pltpu.sync_copy(lab_hbm.at[pl.ds(base, _NPT)], lab_v)
    pltpu.sync_copy(deg_hbm.at[pl.ds(base, _NPT)], deg_v)
    pltpu.sync_copy(bat_hbm.at[pl.ds(base, _NPT)], bat_v)
    ones = jnp.ones((16,), jnp.float32)
    for i in range(_NPT // 16):
        b16 = bat_v[pl.ds(i * 16, 16)] * 128
        l16 = jnp.clip(lab_v[pl.ds(i * 16, 16)], 0, _LV - 1)
        d16 = jnp.clip(deg_v[pl.ds(i * 16, 16)], 0, _DV - 1)
        plsc.addupdate_scatter(tab_v, [b16 + l16], ones)
        plsc.addupdate_scatter(tab_v, [b16 + d16 + 2048], ones)
    pltpu.sync_copy(tab_v, cnt_hbm.at[wid])


def _sc_hist(lab, deg, bat):
    mesh = plsc.VectorSubcoreMesh(core_axis_name="c", subcore_axis_name="s",
                                  num_cores=2, num_subcores=16)
    return pl.kernel(
        _sc_hist_body,
        out_type=jax.ShapeDtypeStruct((_NTILES, _NBINS), jnp.float32),
        mesh=mesh,
        compiler_params=pltpu.CompilerParams(needs_layout_passes=False),
        scratch_types=[
            pltpu.VMEM((_NPT,), jnp.int32),
            pltpu.VMEM((_NPT,), jnp.int32),
            pltpu.VMEM((_NPT,), jnp.int32),
            pltpu.VMEM((_NBINS,), jnp.float32),
        ],
    )(lab, deg, bat)


def _tc_head_body(cnt_ref, gf_ref, le_ref, de_ref, gw_ref, gb_ref,
                  w1_ref, b1_ref, w2_ref, b2_ref, out_ref):
    c = jnp.sum(cnt_ref[...], axis=0)
    counts_lab = c[:_NG, :_LV]
    counts_deg = c[_NG:, :_DV]
    hg = (jnp.dot(counts_lab, le_ref[...], preferred_element_type=jnp.float32,
                  precision=lax.Precision.HIGHEST)
          + jnp.dot(counts_deg, de_ref[...], preferred_element_type=jnp.float32,
                    precision=lax.Precision.HIGHEST))
    gp = jnp.dot(gf_ref[...], gw_ref[...], preferred_element_type=jnp.float32,
                 precision=lax.Precision.HIGHEST) + gb_ref[...]
    cat = jnp.concatenate([hg, gp], axis=1)
    hidden = jnp.maximum(
        jnp.dot(cat, w1_ref[...], preferred_element_type=jnp.float32,
                precision=lax.Precision.HIGHEST) + b1_ref[...], 0.0)
    out_ref[...] = jnp.dot(hidden, w2_ref[...], preferred_element_type=jnp.float32,
                           precision=lax.Precision.HIGHEST) + b2_ref[...]


def kernel(x, batch, g_feat, label_emb, deg_emb, gproj_w, gproj_b,
           w1, b1, w2, b2):
    xi = x.astype(jnp.int32)
    bat = batch.astype(jnp.int32)
    counts = _sc_hist(xi[:, 0], xi[:, 1], bat).reshape(_NTILES, 32, 128)
    out = pl.pallas_call(
        _tc_head_body,
        out_shape=jax.ShapeDtypeStruct((_NG, _NREL), jnp.float32),
    )(counts, g_feat, label_emb, deg_emb, gproj_w, gproj_b.reshape(1, _HIDDEN),
      w1, b1.reshape(1, _HIDDEN), w2, b2.reshape(1, _NREL))
    return out


# speedup vs baseline: 1.7562x; 1.0754x over previous
import jax
import jax.numpy as jnp
from jax import lax
from jax.experimental import pallas as pl
from jax.experimental.pallas import tpu as pltpu
from jax.experimental.pallas import tpu_sc as plsc

_HIDDEN = 128
_NG = 16
_LV = 102
_DV = 11
_NN = 32768
_NREL = 18

_NTILES = 32
_NPT = _NN // _NTILES
_NBINS = 4096


def _sc_hist_body(lab_hbm, deg_hbm, bat_hbm, cnt_hbm, lab_v, deg_v, bat_v, tab_v):
    cid = lax.axis_index("c")
    sid = lax.axis_index("s")
    wid = cid * 16 + sid
    base = wid * _NPT
    for r in range(32):
        for k in range(8):
            tab_v[r, pl.ds(k * 16, 16)] = jnp.zeros((16,), jnp.float32)
    pltpu.sync_copy(lab_hbm.at[pl.ds(base, _NPT)], lab_v)
    pltpu.sync_copy(deg_hbm.at[pl.ds(base, _NPT)], deg_v)
    pltpu.sync_copy(bat_hbm.at[pl.ds(base, _NPT)], bat_v)
    ones = jnp.ones((16,), jnp.float32)
    for i in range(_NPT // 16):
        b16 = bat_v[pl.ds(i * 16, 16)]
        l16 = jnp.clip(lab_v[pl.ds(i * 16, 16)], 0, _LV - 1)
        d16 = jnp.clip(deg_v[pl.ds(i * 16, 16)], 0, _DV - 1)
        plsc.addupdate_scatter(tab_v, [b16, l16], ones)
        plsc.addupdate_scatter(tab_v, [b16 + 16, d16], ones)
    pltpu.sync_copy(tab_v, cnt_hbm.at[wid])


def _sc_hist(lab, deg, bat):
    mesh = plsc.VectorSubcoreMesh(core_axis_name="c", subcore_axis_name="s",
                                  num_cores=2, num_subcores=16)
    return pl.kernel(
        _sc_hist_body,
        out_type=jax.ShapeDtypeStruct((_NTILES, 32, 128), jnp.float32),
        mesh=mesh,
        compiler_params=pltpu.CompilerParams(needs_layout_passes=False),
        scratch_types=[
            pltpu.VMEM((_NPT,), jnp.int32),
            pltpu.VMEM((_NPT,), jnp.int32),
            pltpu.VMEM((_NPT,), jnp.int32),
            pltpu.VMEM((32, 128), jnp.float32),
        ],
    )(lab, deg, bat)


def _tc_head_body(cnt_ref, gf_ref, le_ref, de_ref, gw_ref, gb_ref,
                  w1_ref, b1_ref, w2_ref, b2_ref, out_ref):
    c = jnp.sum(cnt_ref[...], axis=0)
    counts_lab = c[:_NG, :_LV]
    counts_deg = c[_NG:, :_DV]
    hg = (jnp.dot(counts_lab, le_ref[...], preferred_element_type=jnp.float32,
                  precision=lax.Precision.HIGHEST)
          + jnp.dot(counts_deg, de_ref[...], preferred_element_type=jnp.float32,
                    precision=lax.Precision.HIGHEST))
    gp = jnp.dot(gf_ref[...], gw_ref[...], preferred_element_type=jnp.float32,
                 precision=lax.Precision.HIGHEST) + gb_ref[...]
    cat = jnp.concatenate([hg, gp], axis=1)
    hidden = jnp.maximum(
        jnp.dot(cat, w1_ref[...], preferred_element_type=jnp.float32,
                precision=lax.Precision.HIGHEST) + b1_ref[...], 0.0)
    out_ref[...] = jnp.dot(hidden, w2_ref[...], preferred_element_type=jnp.float32,
                           precision=lax.Precision.HIGHEST) + b2_ref[...]


def kernel(x, batch, g_feat, label_emb, deg_emb, gproj_w, gproj_b,
           w1, b1, w2, b2):
    xi = x.astype(jnp.int32)
    bat = batch.astype(jnp.int32)
    counts = _sc_hist(xi[:, 0], xi[:, 1], bat)
    out = pl.pallas_call(
        _tc_head_body,
        out_shape=jax.ShapeDtypeStruct((_NG, _NREL), jnp.float32),
    )(counts, g_feat, label_emb, deg_emb, gproj_w, gproj_b.reshape(1, _HIDDEN),
      w1, b1.reshape(1, _HIDDEN), w2, b2.reshape(1, _NREL))
    return out


# speedup vs baseline: 1.8914x; 1.0770x over previous
import jax
import jax.numpy as jnp
from jax import lax
from jax.experimental import pallas as pl
from jax.experimental.pallas import tpu as pltpu
from jax.experimental.pallas import tpu_sc as plsc

_HIDDEN = 128
_NG = 16
_LV = 102
_DV = 11
_NN = 32768
_NREL = 18

_NTILES = 32
_NPT = _NN // _NTILES
_NBINS = 4096


def _sc_hist_body(lab_hbm, deg_hbm, bat_hbm, cnt_hbm, lab_v, deg_v, bat_v, tab_v):
    cid = lax.axis_index("c")
    sid = lax.axis_index("s")
    wid = cid * 16 + sid
    base = wid * _NPT
    zeros = jnp.zeros((16,), jnp.float32)

    @pl.loop(0, 32)
    def _zero_row(r):
        @pl.loop(0, 8)
        def _zero_chunk(k):
            tab_v[r, pl.ds(k * 16, 16)] = zeros
    pltpu.sync_copy(lab_hbm.at[pl.ds(base, _NPT)], lab_v)
    pltpu.sync_copy(deg_hbm.at[pl.ds(base, _NPT)], deg_v)
    pltpu.sync_copy(bat_hbm.at[pl.ds(base, _NPT)], bat_v)
    ones = jnp.ones((16,), jnp.float32)

    @pl.loop(0, _NPT, step=16)
    def _hist(i):
        b16 = bat_v[pl.ds(i, 16)]
        l16 = jnp.clip(lab_v[pl.ds(i, 16)], 0, _LV - 1)
        d16 = jnp.clip(deg_v[pl.ds(i, 16)], 0, _DV - 1)
        plsc.addupdate_scatter(tab_v, [b16, l16], ones)
        plsc.addupdate_scatter(tab_v, [b16 + 16, d16], ones)
    pltpu.sync_copy(tab_v, cnt_hbm.at[wid])


def _sc_hist(lab, deg, bat):
    mesh = plsc.VectorSubcoreMesh(core_axis_name="c", subcore_axis_name="s",
                                  num_cores=2, num_subcores=16)
    return pl.kernel(
        _sc_hist_body,
        out_type=jax.ShapeDtypeStruct((_NTILES, 32, 128), jnp.float32),
        mesh=mesh,
        compiler_params=pltpu.CompilerParams(needs_layout_passes=False),
        scratch_types=[
            pltpu.VMEM((_NPT,), jnp.int32),
            pltpu.VMEM((_NPT,), jnp.int32),
            pltpu.VMEM((_NPT,), jnp.int32),
            pltpu.VMEM((32, 128), jnp.float32),
        ],
    )(lab, deg, bat)


def _tc_head_body(cnt_ref, gf_ref, le_ref, de_ref, gw_ref, gb_ref,
                  w1_ref, b1_ref, w2_ref, b2_ref, out_ref):
    c = jnp.sum(cnt_ref[...], axis=0)
    counts_lab = c[:_NG, :_LV]
    counts_deg = c[_NG:, :_DV]
    hg = (jnp.dot(counts_lab, le_ref[...], preferred_element_type=jnp.float32,
                  precision=lax.Precision.HIGHEST)
          + jnp.dot(counts_deg, de_ref[...], preferred_element_type=jnp.float32,
                    precision=lax.Precision.HIGHEST))
    gp = jnp.dot(gf_ref[...], gw_ref[...], preferred_element_type=jnp.float32,
                 precision=lax.Precision.HIGHEST) + gb_ref[...]
    cat = jnp.concatenate([hg, gp], axis=1)
    hidden = jnp.maximum(
        jnp.dot(cat, w1_ref[...], preferred_element_type=jnp.float32,
                precision=lax.Precision.HIGHEST) + b1_ref[...], 0.0)
    out_ref[...] = jnp.dot(hidden, w2_ref[...], preferred_element_type=jnp.float32,
                           precision=lax.Precision.HIGHEST) + b2_ref[...]


def kernel(x, batch, g_feat, label_emb, deg_emb, gproj_w, gproj_b,
           w1, b1, w2, b2):
    xi = x.astype(jnp.int32)
    bat = batch.astype(jnp.int32)
    counts = _sc_hist(xi[:, 0], xi[:, 1], bat)
    out = pl.pallas_call(
        _tc_head_body,
        out_shape=jax.ShapeDtypeStruct((_NG, _NREL), jnp.float32),
    )(counts, g_feat, label_emb, deg_emb, gproj_w, gproj_b.reshape(1, _HIDDEN),
      w1, b1.reshape(1, _HIDDEN), w2, b2.reshape(1, _NREL))
    return out
